# Initial kernel scaffold; baseline (speedup 1.0000x reference)
import jax
import jax.numpy as jnp
from jax.experimental import pallas as pl


def kernel(state, step_num, sp_W1, sp_b1, sp_W2, sp_b2, tp_W1, tp_b1, tp_W2, tp_b2, ch_W1, ch_b1, ch_W2, ch_b2, dir_W, dir_b, ss_W, ss_b, v_W, v_b, memory_bank, memory_values):
    raise NotImplementedError("write your pallas kernel here")


# speedup vs baseline: 1.2809x; 1.2809x over previous
import jax
import jax.numpy as jnp
from jax import lax
from jax.experimental import pallas as pl
from jax.experimental.pallas import tpu as pltpu

HID = 4096
H4 = 1024
RD = 256
MEMN = 500
MEMP = 512
TOPK = 3
FUSION = 0.5

NBLK_A = 8
BA = H4 // NBLK_A
NBLK_B = 8
BB = HID // NBLK_B

_PS = lax.Precision.DEFAULT
_PD = lax.Precision.DEFAULT

_NEG = float("-inf")


def _dotT(a, b, prec):
    return lax.dot_general(a, b, (((1,), (1,)), ((), ())),
                           precision=prec, preferred_element_type=jnp.float32)


def _phase_a(state_ref, w1_ref, b1_ref, w2_ref, b2r_ref, bank_ref, mv_ref,
             chw1_ref, chb1_ref, chw2_ref, chb2_ref, dirw_ref, dirb_ref,
             ssw_ref, ssb_ref, vw_ref, vb_ref, tpw1_ref, tpb1_ref,
             g_ref, np_ref, p0_ref, act_ref, lp_ref, val_ref, ent_ref,
             nbank_ref, nvals_ref, rs_s):
    i = pl.program_id(0)
    part = _dotT(state_ref[...], w1_ref[...], _PS) + b1_ref[...]
    h = jnp.maximum(part, 0.0)
    contrib = _dotT(h, w2_ref[...], _PS)

    @pl.when(i == 0)
    def _init():
        rs_s[...] = contrib + b2r_ref[...]

    @pl.when(i > 0)
    def _acc():
        rs_s[...] += contrib

    @pl.when(i == NBLK_A - 1)
    def _epilogue():
        rs = rs_s[...]
        nrm = jnp.sqrt(jnp.sum(rs * rs, axis=1, keepdims=True))
        ns = rs / jnp.maximum(nrm, 1e-12)
        bk = bank_ref[...]
        bnrm = jnp.sqrt(jnp.sum(bk * bk, axis=1, keepdims=True))
        nb = bk / jnp.maximum(bnrm, 1e-12)
        sims = _dotT(ns, nb, _PS)
        ws = sims * (mv_ref[...] + 1e-8)
        li = lax.broadcasted_iota(jnp.int32, (1024, MEMP), 1)
        ws = jnp.where(li >= MEMN, _NEG, ws)
        three = jnp.zeros((1024, MEMP), jnp.float32)
        cur = ws
        for _ in range(TOPK):
            m = jnp.max(cur, axis=1, keepdims=True)
            eq = cur == m
            ii = jnp.min(jnp.where(eq, li, MEMP), axis=1, keepdims=True)
            sel = li == ii
            three = three + sel.astype(jnp.float32)
            cur = jnp.where(sel, _NEG, cur)
        avg = _dotT(three, bk.T, lax.Precision.HIGHEST) / 3.0
        rs_f = (1.0 - FUSION) * rs + FUSION * avg
        c1 = jnp.maximum(_dotT(rs_f, chw1_ref[...], _PS) + chb1_ref[...], 0.0)
        logits = _dotT(c1, chw2_ref[...], _PS) + chb2_ref[...]
        mx = jnp.max(logits, axis=1, keepdims=True)
        e = jnp.exp(logits - mx)
        p = e / jnp.sum(e, axis=1, keepdims=True)
        p0 = p[:, 0:1]
        p1 = p[:, 1:2]
        act = (p1 > p0).astype(jnp.int32)
        p0_ref[...] = p0
        act_ref[...] = act
        lp_ref[...] = jnp.log(jnp.where(act > 0, p1, p0))
        ent_ref[...] = -(p0 * jnp.log(p0 + 1e-8) + p1 * jnp.log(p1 + 1e-8))
        d0 = _dotT(rs_f, dirw_ref[...], _PD) + dirb_ref[...]
        dnrm = jnp.sqrt(jnp.sum(d0 * d0, axis=1, keepdims=True))
        dn = d0 / jnp.maximum(dnrm, 1e-12)
        ssz = jnp.sum(rs_f * ssw_ref[...], axis=1, keepdims=True) + ssb_ref[...]
        ssz = 2.0 / (1.0 + jnp.exp(-ssz))
        val = jnp.sum(rs_f * vw_ref[...], axis=1, keepdims=True) + vb_ref[...]
        val_ref[...] = val
        npos = rs_f + ssz * dn
        np_ref[...] = npos
        pos_mean = jnp.sum(npos, axis=0, keepdims=True) / 1024.0
        val_mean = jnp.sum(val) / 1024.0
        r0 = lax.broadcasted_iota(jnp.int32, (MEMP, RD), 0) == 0
        nbank_ref[...] = jnp.where(r0, pos_mean, bk)
        c0 = lax.broadcasted_iota(jnp.int32, (1, MEMP), 1) == 0
        nvals_ref[...] = jnp.where(c0, val_mean, mv_ref[...])
        g_ref[...] = jnp.maximum(_dotT(npos, tpw1_ref[...], _PD) + tpb1_ref[...], 0.0)


def _phase_b(g_ref, w2_ref, b2_ref, out_ref):
    out_ref[...] = _dotT(g_ref[...], w2_ref[...], _PD) + b2_ref[...]


def kernel(state, step_num, sp_W1, sp_b1, sp_W2, sp_b2, tp_W1, tp_b1, tp_W2,
           tp_b2, ch_W1, ch_b1, ch_W2, ch_b2, dir_W, dir_b, ss_W, ss_b, v_W,
           v_b, memory_bank, memory_values):
    f32 = jnp.float32
    se = jnp.sin(jnp.asarray(step_num, f32) * 0.5)
    b2r = (sp_b2 + 0.1 * se).reshape(1, RD)
    bank_p = jnp.zeros((MEMP, RD), f32).at[:MEMN].set(memory_bank)
    mv_p = jnp.zeros((1, MEMP), f32).at[0, :MEMN].set(memory_values)

    const = lambda shape: pl.BlockSpec(shape, lambda i: (0,) * len(shape))
    outs_a = (
        jax.ShapeDtypeStruct((1024, H4), f32),
        jax.ShapeDtypeStruct((1024, RD), f32),
        jax.ShapeDtypeStruct((1024, 1), f32),
        jax.ShapeDtypeStruct((1024, 1), jnp.int32),
        jax.ShapeDtypeStruct((1024, 1), f32),
        jax.ShapeDtypeStruct((1024, 1), f32),
        jax.ShapeDtypeStruct((1024, 1), f32),
        jax.ShapeDtypeStruct((MEMP, RD), f32),
        jax.ShapeDtypeStruct((1, MEMP), f32),
    )
    (g, npos, p0, act, lp, val, ent, nbank, nvals) = pl.pallas_call(
        _phase_a,
        grid=(NBLK_A,),
        in_specs=[
            const((1024, HID)),
            pl.BlockSpec((BA, HID), lambda i: (i, 0)),
            pl.BlockSpec((1, BA), lambda i: (0, i)),
            pl.BlockSpec((RD, BA), lambda i: (0, i)),
            const((1, RD)),
            const((MEMP, RD)),
            const((1, MEMP)),
            const((128, RD)),
            const((1, 128)),
            const((2, 128)),
            const((1, 2)),
            const((RD, RD)),
            const((1, RD)),
            const((1, RD)),
            const((1, 1)),
            const((1, RD)),
            const((1, 1)),
            const((H4, RD)),
            const((1, H4)),
        ],
        out_specs=[
            const((1024, H4)),
            const((1024, RD)),
            const((1024, 1)),
            const((1024, 1)),
            const((1024, 1)),
            const((1024, 1)),
            const((1024, 1)),
            const((MEMP, RD)),
            const((1, MEMP)),
        ],
        out_shape=outs_a,
        scratch_shapes=[pltpu.VMEM((1024, RD), f32)],
        compiler_params=pltpu.CompilerParams(
            dimension_semantics=("arbitrary",)),
    )(state, sp_W1, sp_b1.reshape(1, H4), sp_W2, b2r, bank_p, mv_p,
      ch_W1, ch_b1.reshape(1, 128), ch_W2, ch_b2.reshape(1, 2),
      dir_W, dir_b.reshape(1, RD), ss_W, ss_b.reshape(1, 1),
      v_W, v_b.reshape(1, 1), tp_W1, tp_b1.reshape(1, H4))

    latent = pl.pallas_call(
        _phase_b,
        grid=(NBLK_B,),
        in_specs=[
            const((1024, H4)),
            pl.BlockSpec((BB, H4), lambda i: (i, 0)),
            pl.BlockSpec((1, BB), lambda i: (0, i)),
        ],
        out_specs=pl.BlockSpec((1024, BB), lambda i: (0, i)),
        out_shape=jax.ShapeDtypeStruct((1024, HID), f32),
        compiler_params=pltpu.CompilerParams(
            dimension_semantics=("arbitrary",)),
    )(g, tp_W2, tp_b2.reshape(1, HID))

    return (latent, npos, p0[:, 0], act[:, 0], lp[:, 0], val[:, 0],
            ent[:, 0], nbank[:MEMN], nvals[0, :MEMN])


# speedup vs baseline: 1.7824x; 1.3915x over previous
import jax
import jax.numpy as jnp
from jax import lax
from jax.experimental import pallas as pl
from jax.experimental.pallas import tpu as pltpu

HID = 4096
H4 = 1024
RD = 256
MEMN = 500
TOPK = 3
FUSION = 0.5

NK = 8
BK = HID // NK
NB = 8
BB = HID // NB

_PS = lax.Precision.DEFAULT
_PD = lax.Precision.DEFAULT

_NEG = float("-inf")


def _dotT(a, b, prec):
    return lax.dot_general(a, b, (((1,), (1,)), ((), ())),
                           precision=prec, preferred_element_type=jnp.float32)


def _body(state_ref, w1_ref, b1_ref, w2_ref, b2r_ref, bank_ref, mv_ref,
          chw1_ref, chb1_ref, chw2_ref, chb2_ref, dirw_ref, dirb_ref,
          ssw_ref, ssb_ref, vw_ref, vb_ref, tpw1_ref, tpb1_ref,
          tpw2_ref, tpb2_ref,
          lat_ref, np_ref, p0_ref, act_ref, lp_ref, val_ref, ent_ref,
          nbank_ref, nvals_ref, h_s, g_s):
    i = pl.program_id(0)

    @pl.when(i == 0)
    def _k0():
        h_s[...] = _dotT(state_ref[...], w1_ref[...], _PS)

    @pl.when((i > 0) & (i < NK))
    def _kacc():
        h_s[...] += _dotT(state_ref[...], w1_ref[...], _PS)

    @pl.when(i == NK - 1)
    def _epilogue():
        h = jnp.maximum(h_s[...] + b1_ref[...], 0.0)
        rs = _dotT(h, w2_ref[...], _PS) + b2r_ref[...]
        nrm = jnp.sqrt(jnp.sum(rs * rs, axis=1, keepdims=True))
        ns = rs / jnp.maximum(nrm, 1e-12)
        bk = bank_ref[...]
        bnrm = jnp.sqrt(jnp.sum(bk * bk, axis=1, keepdims=True))
        nb = bk / jnp.maximum(bnrm, 1e-12)
        sims = _dotT(ns, nb, _PS)
        ws = sims * (mv_ref[...] + 1e-8)
        li = lax.broadcasted_iota(jnp.int32, (1024, MEMN), 1)
        three = jnp.zeros((1024, MEMN), jnp.float32)
        cur = ws
        for _ in range(TOPK):
            m = jnp.max(cur, axis=1, keepdims=True)
            eq = cur == m
            ii = jnp.min(jnp.where(eq, li, MEMN), axis=1, keepdims=True)
            sel = li == ii
            three = three + sel.astype(jnp.float32)
            cur = jnp.where(sel, _NEG, cur)
        avg = _dotT(three, bk.T, lax.Precision.HIGHEST) / 3.0
        rs_f = (1.0 - FUSION) * rs + FUSION * avg
        c1 = jnp.maximum(_dotT(rs_f, chw1_ref[...], _PS) + chb1_ref[...], 0.0)
        logits = _dotT(c1, chw2_ref[...], _PS) + chb2_ref[...]
        mx = jnp.max(logits, axis=1, keepdims=True)
        e = jnp.exp(logits - mx)
        p = e / jnp.sum(e, axis=1, keepdims=True)
        p0 = p[:, 0:1]
        p1 = p[:, 1:2]
        act = (p1 > p0).astype(jnp.int32)
        p0_ref[...] = p0
        act_ref[...] = act
        lp_ref[...] = jnp.log(jnp.where(act > 0, p1, p0))
        ent_ref[...] = -(p0 * jnp.log(p0 + 1e-8) + p1 * jnp.log(p1 + 1e-8))
        d0 = _dotT(rs_f, dirw_ref[...], _PD) + dirb_ref[...]
        dnrm = jnp.sqrt(jnp.sum(d0 * d0, axis=1, keepdims=True))
        dn = d0 / jnp.maximum(dnrm, 1e-12)
        ssz = jnp.sum(rs_f * ssw_ref[...], axis=1, keepdims=True) + ssb_ref[...]
        ssz = 2.0 / (1.0 + jnp.exp(-ssz))
        val = jnp.sum(rs_f * vw_ref[...], axis=1, keepdims=True) + vb_ref[...]
        val_ref[...] = val
        npos = rs_f + ssz * dn
        np_ref[...] = npos
        pos_mean = jnp.sum(npos, axis=0, keepdims=True) / 1024.0
        val_mean = jnp.sum(val) / 1024.0
        r0 = lax.broadcasted_iota(jnp.int32, (MEMN, RD), 0) == 0
        nbank_ref[...] = jnp.where(r0, pos_mean, bk)
        c0 = lax.broadcasted_iota(jnp.int32, (1, MEMN), 1) == 0
        nvals_ref[...] = jnp.where(c0, val_mean, mv_ref[...])
        g_s[...] = jnp.maximum(_dotT(npos, tpw1_ref[...], _PD) + tpb1_ref[...],
                               0.0)

    @pl.when(i >= NK)
    def _phase_b():
        lat_ref[...] = _dotT(g_s[...], tpw2_ref[...], _PD) + tpb2_ref[...]


def kernel(state, step_num, sp_W1, sp_b1, sp_W2, sp_b2, tp_W1, tp_b1, tp_W2,
           tp_b2, ch_W1, ch_b1, ch_W2, ch_b2, dir_W, dir_b, ss_W, ss_b, v_W,
           v_b, memory_bank, memory_values):
    f32 = jnp.float32
    se = jnp.sin(jnp.asarray(step_num, f32) * 0.5)
    b2r = (sp_b2 + 0.1 * se).reshape(1, RD)

    const = lambda shape: pl.BlockSpec(shape, lambda i: (0,) * len(shape))
    outs = (
        jax.ShapeDtypeStruct((1024, HID), f32),
        jax.ShapeDtypeStruct((1024, RD), f32),
        jax.ShapeDtypeStruct((1024, 1), f32),
        jax.ShapeDtypeStruct((1024, 1), jnp.int32),
        jax.ShapeDtypeStruct((1024, 1), f32),
        jax.ShapeDtypeStruct((1024, 1), f32),
        jax.ShapeDtypeStruct((1024, 1), f32),
        jax.ShapeDtypeStruct((MEMN, RD), f32),
        jax.ShapeDtypeStruct((1, MEMN), f32),
    )
    ka = NK - 1
    (lat, npos, p0, act, lp, val, ent, nbank, nvals) = pl.pallas_call(
        _body,
        grid=(NK + NB,),
        in_specs=[
            pl.BlockSpec((1024, BK), lambda i: (0, jnp.minimum(i, ka))),
            pl.BlockSpec((1024, BK), lambda i: (0, jnp.minimum(i, ka))),
            const((1, H4)),
            const((RD, H4)),
            const((1, RD)),
            const((MEMN, RD)),
            const((1, MEMN)),
            const((128, RD)),
            const((1, 128)),
            const((2, 128)),
            const((1, 2)),
            const((RD, RD)),
            const((1, RD)),
            const((1, RD)),
            const((1, 1)),
            const((1, RD)),
            const((1, 1)),
            const((H4, RD)),
            const((1, H4)),
            pl.BlockSpec((BB, H4), lambda i: (jnp.maximum(i - NK, 0), 0)),
            pl.BlockSpec((1, BB), lambda i: (0, jnp.maximum(i - NK, 0))),
        ],
        out_specs=[
            pl.BlockSpec((1024, BB), lambda i: (0, jnp.maximum(i - NK, 0))),
            const((1024, RD)),
            const((1024, 1)),
            const((1024, 1)),
            const((1024, 1)),
            const((1024, 1)),
            const((1024, 1)),
            const((MEMN, RD)),
            const((1, MEMN)),
        ],
        out_shape=outs,
        scratch_shapes=[pltpu.VMEM((1024, H4), f32),
                        pltpu.VMEM((1024, H4), f32)],
        compiler_params=pltpu.CompilerParams(
            dimension_semantics=("arbitrary",)),
    )(state, sp_W1, sp_b1.reshape(1, H4), sp_W2, b2r, memory_bank,
      memory_values.reshape(1, MEMN),
      ch_W1, ch_b1.reshape(1, 128), ch_W2, ch_b2.reshape(1, 2),
      dir_W, dir_b.reshape(1, RD), ss_W, ss_b.reshape(1, 1),
      v_W, v_b.reshape(1, 1), tp_W1, tp_b1.reshape(1, H4),
      tp_W2, tp_b2.reshape(1, HID))

    return (lat, npos, p0[:, 0], act[:, 0], lp[:, 0], val[:, 0],
            ent[:, 0], nbank, nvals[0])


# speedup vs baseline: 2.0168x; 1.1315x over previous
import jax
import jax.numpy as jnp
from jax import lax
from jax.experimental import pallas as pl
from jax.experimental.pallas import tpu as pltpu

HID = 4096
H4 = 1024
RD = 256
MEMN = 500
TOPK = 3
FUSION = 0.5

NK = 8
BK = HID // NK
NB = 8
BB = HID // NB

_PS = lax.Precision.DEFAULT
_PD = lax.Precision.DEFAULT

_NEG = float("-inf")


def _dotT(a, b, prec):
    return lax.dot_general(a, b, (((1,), (1,)), ((), ())),
                           precision=prec, preferred_element_type=jnp.float32)


def _body(state_ref, w1_ref, b1_ref, w2_ref, b2r_ref, bank_ref, mv_ref,
          chw1_ref, chb1_ref, chw2_ref, chb2_ref, dirw_ref, dirb_ref,
          ssw_ref, ssb_ref, vw_ref, vb_ref, tpw1_ref, tpb1_ref,
          tpw2_ref, tpb2_ref,
          lat_ref, np_ref, p0_ref, act_ref, lp_ref, val_ref, ent_ref,
          nbank_ref, nvals_ref, h_s, g_s):
    i = pl.program_id(0)

    @pl.when(i == 0)
    def _k0():
        h_s[...] = _dotT(state_ref[...], w1_ref[...], _PS)

    @pl.when((i > 0) & (i < NK))
    def _kacc():
        h_s[...] += _dotT(state_ref[...], w1_ref[...], _PS)

    @pl.when(i == NK - 1)
    def _epilogue():
        h = jnp.maximum(h_s[...] + b1_ref[...].reshape(1, H4), 0.0)
        rs = _dotT(h, w2_ref[...], _PS) + b2r_ref[...].reshape(1, RD)
        nrm = jnp.sqrt(jnp.sum(rs * rs, axis=1, keepdims=True))
        ns = rs / jnp.maximum(nrm, 1e-12)
        bk = bank_ref[...]
        bnrm = jnp.sqrt(jnp.sum(bk * bk, axis=1, keepdims=True))
        nb = bk / jnp.maximum(bnrm, 1e-12)
        sims = _dotT(ns, nb, _PS)
        mv = mv_ref[...].reshape(1, MEMN)
        ws = sims * (mv + 1e-8)
        li = lax.broadcasted_iota(jnp.int32, (1024, MEMN), 1)
        three = jnp.zeros((1024, MEMN), jnp.float32)
        cur = ws
        for _ in range(TOPK):
            m = jnp.max(cur, axis=1, keepdims=True)
            eq = cur == m
            ii = jnp.min(jnp.where(eq, li, MEMN), axis=1, keepdims=True)
            sel = li == ii
            three = three + sel.astype(jnp.float32)
            cur = jnp.where(sel, _NEG, cur)
        avg = _dotT(three, bk.T, lax.Precision.HIGHEST) / 3.0
        rs_f = (1.0 - FUSION) * rs + FUSION * avg
        c1 = jnp.maximum(_dotT(rs_f, chw1_ref[...], _PS) + chb1_ref[...].reshape(1, 128), 0.0)
        logits = _dotT(c1, chw2_ref[...], _PS) + chb2_ref[...].reshape(1, 2)
        mx = jnp.max(logits, axis=1, keepdims=True)
        e = jnp.exp(logits - mx)
        p = e / jnp.sum(e, axis=1, keepdims=True)
        p0 = p[:, 0:1]
        p1 = p[:, 1:2]
        act = (p1 > p0).astype(jnp.int32)
        p0_ref[...] = p0.T
        act_ref[...] = act.T
        lp_ref[...] = jnp.log(jnp.where(act > 0, p1, p0)).T
        ent_ref[...] = (-(p0 * jnp.log(p0 + 1e-8) + p1 * jnp.log(p1 + 1e-8))).T
        d0 = _dotT(rs_f, dirw_ref[...], _PD) + dirb_ref[...].reshape(1, RD)
        dnrm = jnp.sqrt(jnp.sum(d0 * d0, axis=1, keepdims=True))
        dn = d0 / jnp.maximum(dnrm, 1e-12)
        ssz = jnp.sum(rs_f * ssw_ref[...], axis=1, keepdims=True) + ssb_ref[...].reshape(1, 1)
        ssz = 2.0 / (1.0 + jnp.exp(-ssz))
        val = jnp.sum(rs_f * vw_ref[...], axis=1, keepdims=True) + vb_ref[...].reshape(1, 1)
        val_ref[...] = val.T
        npos = rs_f + ssz * dn
        np_ref[...] = npos
        pos_mean = jnp.sum(npos, axis=0, keepdims=True) / 1024.0
        val_mean = jnp.sum(val) / 1024.0
        r0 = lax.broadcasted_iota(jnp.int32, (MEMN, RD), 0) == 0
        nbank_ref[...] = jnp.where(r0, pos_mean, bk)
        c0 = lax.broadcasted_iota(jnp.int32, (1, MEMN), 1) == 0
        nvals_ref[...] = jnp.where(c0, val_mean, mv)
        g_s[...] = jnp.maximum(_dotT(npos, tpw1_ref[...], _PD) + tpb1_ref[...].reshape(1, H4),
                               0.0)

    @pl.when(i >= NK)
    def _phase_b():
        lat_ref[...] = _dotT(g_s[...], tpw2_ref[...], _PD) + tpb2_ref[...].reshape(1, BB)


def kernel(state, step_num, sp_W1, sp_b1, sp_W2, sp_b2, tp_W1, tp_b1, tp_W2,
           tp_b2, ch_W1, ch_b1, ch_W2, ch_b2, dir_W, dir_b, ss_W, ss_b, v_W,
           v_b, memory_bank, memory_values):
    f32 = jnp.float32
    se = jnp.sin(jnp.asarray(step_num, f32) * 0.5)
    b2r = sp_b2 + 0.1 * se

    const = lambda shape: pl.BlockSpec(shape, lambda i: (0,) * len(shape))
    outs = (
        jax.ShapeDtypeStruct((1024, HID), f32),
        jax.ShapeDtypeStruct((1024, RD), f32),
        jax.ShapeDtypeStruct((1, 1024), f32),
        jax.ShapeDtypeStruct((1, 1024), jnp.int32),
        jax.ShapeDtypeStruct((1, 1024), f32),
        jax.ShapeDtypeStruct((1, 1024), f32),
        jax.ShapeDtypeStruct((1, 1024), f32),
        jax.ShapeDtypeStruct((MEMN, RD), f32),
        jax.ShapeDtypeStruct((1, MEMN), f32),
    )
    ka = NK - 1
    (lat, npos, p0, act, lp, val, ent, nbank, nvals) = pl.pallas_call(
        _body,
        grid=(NK + NB,),
        in_specs=[
            pl.BlockSpec((1024, BK), lambda i: (0, jnp.minimum(i, ka))),
            pl.BlockSpec((1024, BK), lambda i: (0, jnp.minimum(i, ka))),
            const((H4,)),
            const((RD, H4)),
            const((RD,)),
            const((MEMN, RD)),
            const((MEMN,)),
            const((128, RD)),
            const((128,)),
            const((2, 128)),
            const((2,)),
            const((RD, RD)),
            const((RD,)),
            const((1, RD)),
            const((1,)),
            const((1, RD)),
            const((1,)),
            const((H4, RD)),
            const((H4,)),
            pl.BlockSpec((BB, H4), lambda i: (jnp.maximum(i - NK, 0), 0)),
            pl.BlockSpec((BB,), lambda i: (jnp.maximum(i - NK, 0),)),
        ],
        out_specs=[
            pl.BlockSpec((1024, BB), lambda i: (0, jnp.maximum(i - NK, 0))),
            const((1024, RD)),
            const((1, 1024)),
            const((1, 1024)),
            const((1, 1024)),
            const((1, 1024)),
            const((1, 1024)),
            const((MEMN, RD)),
            const((1, MEMN)),
        ],
        out_shape=outs,
        scratch_shapes=[pltpu.VMEM((1024, H4), f32),
                        pltpu.VMEM((1024, H4), f32)],
        compiler_params=pltpu.CompilerParams(
            dimension_semantics=("arbitrary",)),
    )(state, sp_W1, sp_b1, sp_W2, b2r, memory_bank, memory_values,
      ch_W1, ch_b1, ch_W2, ch_b2, dir_W, dir_b, ss_W, ss_b,
      v_W, v_b, tp_W1, tp_b1, tp_W2, tp_b2)

    return (lat, npos, p0[0], act[0], lp[0], val[0],
            ent[0], nbank, nvals[0])
